# Initial kernel scaffold; baseline (speedup 1.0000x reference)
import jax
import jax.numpy as jnp
from jax.experimental import pallas as pl


def kernel(x, edge_index, edge_attr, W_ih, W_hh, b_ih, b_hh, W1, att_s1, att_d1, We1, att_e1, bias1, gamma1, beta1, W2, att_s2, att_d2, We2, att_e2, bias2, gamma2, beta2, W3, att_s3, att_d3, We3, att_e3, bias3):
    raise NotImplementedError("write your pallas kernel here")


# speedup vs baseline: 12.7194x; 12.7194x over previous
import functools

import jax
import jax.numpy as jnp
from jax import lax
from jax.experimental import pallas as pl
from jax.experimental.pallas import tpu as pltpu
from jax.experimental.pallas import tpu_sc as plsc

N = 100000
E = 1600000
T = 24
CIN = 2
H = 32
COUT = 2
ED = 4

NC = 2
NS = 16
NW = NC * NS

KE = 128
NGRP = KE // 16
TOTG = E // KE
BASEG = TOTG // NW
EXTRAG = TOTG % NW

KN = 5000
NROUND = N // (KN * NS) + (1 if N % (KN * NS) else 0)

NPAD = 100096
SUBROWS = NPAD // NS
WB = 368
NWB = SUBROWS // WB

f32 = jnp.float32


def _mesh():
    return plsc.VectorSubcoreMesh(
        core_axis_name="c", subcore_axis_name="s", num_cores=NC,
        num_subcores=NS)


def _stage_1d(hbm_ref, sp_ref, stage_v, s_i):
    for r in range(2):
        start = (r * NS + s_i) * KN

        @pl.when(start < N)
        def _():
            pltpu.sync_copy(hbm_ref.at[pl.ds(start, KN)],
                            stage_v.at[pl.ds(0, KN)])
            pltpu.sync_copy(stage_v.at[pl.ds(0, KN)],
                            sp_ref.at[pl.ds(start, KN)])


def _zero_1d(sp_ref, zero_v, s_i):
    for r in range(2):
        start = (r * NS + s_i) * KN

        @pl.when(start < N)
        def _():
            pltpu.sync_copy(zero_v.at[pl.ds(0, KN)],
                            sp_ref.at[pl.ds(start, KN)])


def _writeback_1d(sp_ref, hbm_flat, core_off, stage_v, s_i):
    for r in range(2):
        start = (r * NS + s_i) * KN

        @pl.when(start < N)
        def _():
            pltpu.sync_copy(sp_ref.at[pl.ds(start, KN)],
                            stage_v.at[pl.ds(0, KN)])
            pltpu.sync_copy(stage_v.at[pl.ds(0, KN)],
                            hbm_flat.at[pl.ds(core_off + start, KN)])


def _fill_zeros(ref, nelem):
    z = jnp.zeros((16,), f32)

    @pl.loop(0, nelem // 16)
    def _(i):
        ref[pl.ds(i * 16, 16)] = z


def _pa_rows_body(src_hbm, dst_hbm, ea_hbm, hw_hbm, acc_out,
                  sp_acc, v_src, v_dst, v_ea, v_rows, v_zrows, sem_b):
    c_i = lax.axis_index("c")
    s_i = lax.axis_index("s")
    wid = c_i * NS + s_i

    @pl.loop(0, WB)
    def _(i):
        v_zrows[i, :] = jnp.zeros((16,), f32)

    for r in range(NWB):
        row0 = s_i * SUBROWS + r * WB
        pltpu.sync_copy(v_zrows, sp_acc.at[pl.ds(row0, WB)])

    plsc.subcore_barrier()
    start_g = wid * BASEG + jnp.minimum(wid, EXTRAG)
    n_g = BASEG + (wid < EXTRAG).astype(jnp.int32)

    @pl.loop(0, n_g)
    def _chunk(i):
        base = (start_g + i) * KE
        pltpu.sync_copy(src_hbm.at[pl.ds(base, KE)], v_src)
        pltpu.sync_copy(dst_hbm.at[pl.ds(base, KE)], v_dst)
        pltpu.sync_copy(ea_hbm.at[pl.ds(base, KE)], v_ea)
        pltpu.async_copy(hw_hbm.at[v_src], v_rows, sem_b).wait()

        for g in range(NGRP):
            va = v_ea[pl.ds(g * 16, 16)]
            for j in range(16):
                e = g * 16 + j
                v_rows[e, :] = va[j] * v_rows[e, :]

        pltpu.sync_copy(v_rows, sp_acc.at[v_dst], add=True)

    plsc.subcore_barrier()

    for r in range(NWB):
        row0 = s_i * SUBROWS + r * WB
        pltpu.sync_copy(sp_acc.at[pl.ds(row0, WB)], v_zrows)
        pltpu.sync_copy(v_zrows, acc_out.at[c_i, pl.ds(row0, WB)])


@functools.lru_cache(maxsize=None)
def _make_pa_rows():
    return pl.kernel(
        _pa_rows_body,
        out_type=jax.ShapeDtypeStruct((NC, NPAD, 16), f32),
        mesh=_mesh(),
        scratch_types=[
            pltpu.VMEM_SHARED((NPAD, 16), f32),
            pltpu.VMEM((KE,), jnp.int32),
            pltpu.VMEM((KE,), jnp.int32),
            pltpu.VMEM((KE,), f32),
            pltpu.VMEM((KE, 16), f32),
            pltpu.VMEM((WB, 16), f32),
            pltpu.SemaphoreType.DMA,
        ],
        compiler_params=pltpu.CompilerParams(use_tc_tiling_on_sc=False,
                                             needs_layout_passes=False))


def _pa3_ea_body(src_hbm, dst_hbm, c_hbm, ss_hbm, sd_hbm, m_hbm,
                 ea_out, den_out, sp_ss, sp_sd, sp_den,
                 v_src, v_dst, v_c, v_ss, v_sd, v_ea, v_stage, v_m, sem_a):
    c_i = lax.axis_index("c")
    s_i = lax.axis_index("s")
    wid = c_i * NS + s_i

    _fill_zeros(v_stage, KN)
    _stage_1d(ss_hbm, sp_ss, v_stage, s_i)
    _fill_zeros(v_stage, KN)
    _stage_1d(sd_hbm, sp_sd, v_stage, s_i)
    _fill_zeros(v_stage, KN)
    _zero_1d(sp_den, v_stage, s_i)
    pltpu.sync_copy(m_hbm, v_m)
    plsc.subcore_barrier()

    mv = v_m[...]
    start_g = wid * BASEG + jnp.minimum(wid, EXTRAG)
    n_g = BASEG + (wid < EXTRAG).astype(jnp.int32)

    @pl.loop(0, n_g)
    def _chunk(i):
        base = (start_g + i) * KE
        pltpu.sync_copy(src_hbm.at[pl.ds(base, KE)], v_src)
        pltpu.sync_copy(dst_hbm.at[pl.ds(base, KE)], v_dst)
        pltpu.sync_copy(c_hbm.at[pl.ds(base, KE)], v_c)
        pltpu.async_copy(sp_ss.at[v_src], v_ss, sem_a).wait()
        pltpu.async_copy(sp_sd.at[v_dst], v_sd, sem_a).wait()

        for g in range(NGRP):
            sl = pl.ds(g * 16, 16)
            a = v_ss[sl] + v_sd[sl] + v_c[sl]
            a = jnp.where(a < 0, a * 0.2, a)
            v_ea[sl] = jnp.exp(a - mv)

        pltpu.sync_copy(v_ea, ea_out.at[pl.ds(base, KE)])
        pltpu.sync_copy(v_ea, sp_den.at[v_dst], add=True)

    plsc.subcore_barrier()
    _writeback_1d(sp_den, den_out, c_i * N, v_stage, s_i)


@functools.lru_cache(maxsize=None)
def _make_pa3_ea():
  return pl.kernel(
    _pa3_ea_body,
    out_type=(jax.ShapeDtypeStruct((E,), f32),
              jax.ShapeDtypeStruct((NC * N,), f32)),
    mesh=_mesh(),
    scratch_types=[
        pltpu.VMEM_SHARED((N,), f32),
        pltpu.VMEM_SHARED((N,), f32),
        pltpu.VMEM_SHARED((N,), f32),
        pltpu.VMEM((KE,), jnp.int32),
        pltpu.VMEM((KE,), jnp.int32),
        pltpu.VMEM((KE,), f32),
        pltpu.VMEM((KE,), f32),
        pltpu.VMEM((KE,), f32),
        pltpu.VMEM((KE,), f32),
        pltpu.VMEM((KN,), f32),
        pltpu.VMEM((16,), f32),
        pltpu.SemaphoreType.DMA,
    ],
    compiler_params=pltpu.CompilerParams(use_tc_tiling_on_sc=False,
                                             needs_layout_passes=False))


BN = 800
NBN = N // BN
BE = 6400
NBE = E // BE


def _lstm_body(xt_ref, wih_ref, whh_ref, b_ref, out_ref):
    h0 = jnp.zeros((BN, H), f32)
    c0 = jnp.zeros((BN, H), f32)

    def step(t, hc):
        h, c = hc
        g = (jnp.dot(xt_ref[t], wih_ref[...], preferred_element_type=f32)
             + jnp.dot(h, whh_ref[...], preferred_element_type=f32)
             + b_ref[...])
        ig = jax.nn.sigmoid(g[:, 0:H])
        fg = jax.nn.sigmoid(g[:, H:2 * H])
        gg = jnp.tanh(g[:, 2 * H:3 * H])
        og = jax.nn.sigmoid(g[:, 3 * H:4 * H])
        c = fg * c + ig * gg
        h = og * jnp.tanh(c)
        return (h, c)

    h, _ = lax.fori_loop(0, T, step, (h0, c0))
    out_ref[...] = h


_lstm_tc = pl.pallas_call(
    _lstm_body,
    grid=(NBN,),
    in_specs=[
        pl.BlockSpec((T, BN, CIN), lambda i: (0, i, 0)),
        pl.BlockSpec((CIN, 4 * H), lambda i: (0, 0)),
        pl.BlockSpec((H, 4 * H), lambda i: (0, 0)),
        pl.BlockSpec((1, 4 * H), lambda i: (0, 0)),
    ],
    out_specs=pl.BlockSpec((BN, H), lambda i: (i, 0)),
    out_shape=jax.ShapeDtypeStruct((N, H), f32),
    compiler_params=pltpu.CompilerParams(dimension_semantics=("arbitrary",)),
)


def _attr_body(eat_ref, we1_ref, ae1_ref, we2_ref, ae2_ref, we3_ref, ae3_ref,
               c_ref, mx_ref):
    u1 = jnp.sum(we1_ref[...] * ae1_ref[...], axis=1).reshape(1, ED)
    u2 = jnp.sum(we2_ref[...] * ae2_ref[...], axis=1).reshape(1, ED)
    u3 = jnp.sum(we3_ref[...] * ae3_ref[...], axis=1).reshape(1, ED)
    ut = jnp.concatenate([u1, u2, u3], axis=0)
    c = jnp.dot(ut, eat_ref[...], preferred_element_type=f32)
    c_ref[...] = c
    mx_ref[...] = jnp.max(c, axis=1, keepdims=True)[None]


_attr_tc = pl.pallas_call(
    _attr_body,
    grid=(NBE,),
    in_specs=[
        pl.BlockSpec((ED, BE), lambda i: (0, i)),
        pl.BlockSpec((ED, H), lambda i: (0, 0)),
        pl.BlockSpec((1, H), lambda i: (0, 0)),
        pl.BlockSpec((ED, H), lambda i: (0, 0)),
        pl.BlockSpec((1, H), lambda i: (0, 0)),
        pl.BlockSpec((ED, COUT), lambda i: (0, 0)),
        pl.BlockSpec((1, COUT), lambda i: (0, 0)),
    ],
    out_specs=[
        pl.BlockSpec((3, BE), lambda i: (0, i)),
        pl.BlockSpec((1, 3, 1), lambda i: (i, 0, 0)),
    ],
    out_shape=[
        jax.ShapeDtypeStruct((3, E), f32),
        jax.ShapeDtypeStruct((NBE, 3, 1), f32),
    ],
    compiler_params=pltpu.CompilerParams(dimension_semantics=("arbitrary",)),
)


def _proj_block(h, w_ref, as_ref, ad_ref):
    hw = jnp.dot(h, w_ref[...], preferred_element_type=f32)
    ss = jnp.sum(hw * as_ref[...], axis=-1)
    sd = jnp.sum(hw * ad_ref[...], axis=-1)
    return hw, ss, sd


def _gluea_body(h_ref, w_ref, as_ref, ad_ref,
                lo_ref, hi_ref, ss_ref, sd_ref, ms_ref, md_ref):
    hw, ss, sd = _proj_block(h_ref[...], w_ref, as_ref, ad_ref)
    lo_ref[...] = hw[:, :16]
    hi_ref[...] = hw[:, 16:]
    ss_ref[...] = ss[:, None]
    sd_ref[...] = sd[:, None]
    ms_ref[...] = jnp.max(ss).reshape(1, 1, 1)
    md_ref[...] = jnp.max(sd).reshape(1, 1, 1)


_gluea_tc = pl.pallas_call(
    _gluea_body,
    grid=(NBN,),
    in_specs=[
        pl.BlockSpec((BN, H), lambda i: (i, 0)),
        pl.BlockSpec((H, H), lambda i: (0, 0)),
        pl.BlockSpec((1, H), lambda i: (0, 0)),
        pl.BlockSpec((1, H), lambda i: (0, 0)),
    ],
    out_specs=[
        pl.BlockSpec((BN, 16), lambda i: (i, 0)),
        pl.BlockSpec((BN, 16), lambda i: (i, 0)),
        pl.BlockSpec((BN, 1), lambda i: (i, 0)),
        pl.BlockSpec((BN, 1), lambda i: (i, 0)),
        pl.BlockSpec((1, 1, 1), lambda i: (i, 0, 0)),
        pl.BlockSpec((1, 1, 1), lambda i: (i, 0, 0)),
    ],
    out_shape=[
        jax.ShapeDtypeStruct((N, 16), f32),
        jax.ShapeDtypeStruct((N, 16), f32),
        jax.ShapeDtypeStruct((N, 1), f32),
        jax.ShapeDtypeStruct((N, 1), f32),
        jax.ShapeDtypeStruct((NBN, 1, 1), f32),
        jax.ShapeDtypeStruct((NBN, 1, 1), f32),
    ],
    compiler_params=pltpu.CompilerParams(dimension_semantics=("arbitrary",)),
)


def _combine_block(alo_ref, ahi_ref, den_ref, bias_ref, gamma_ref, beta_ref):
    num = jnp.concatenate(
        [alo_ref[0] + alo_ref[1], ahi_ref[0] + ahi_ref[1]], axis=1)
    den = den_ref[0] + den_ref[1]
    g = num / (den + 1e-16) + bias_ref[...]
    mu = jnp.mean(g, axis=-1, keepdims=True)
    var = jnp.mean((g - mu) ** 2, axis=-1, keepdims=True)
    y = (g - mu) / jnp.sqrt(var + 1e-5) * gamma_ref[...] + beta_ref[...]
    return jax.nn.relu(y)


def _glueb12_body(alo_ref, ahi_ref, den_ref, bias_ref, gamma_ref, beta_ref,
                  w_ref, as_ref, ad_ref,
                  lo_ref, hi_ref, ss_ref, sd_ref, ms_ref, md_ref):
    y = _combine_block(alo_ref, ahi_ref, den_ref, bias_ref, gamma_ref,
                       beta_ref)
    hw, ss, sd = _proj_block(y, w_ref, as_ref, ad_ref)
    lo_ref[...] = hw[:, :16]
    hi_ref[...] = hw[:, 16:]
    ss_ref[...] = ss[:, None]
    sd_ref[...] = sd[:, None]
    ms_ref[...] = jnp.max(ss).reshape(1, 1, 1)
    md_ref[...] = jnp.max(sd).reshape(1, 1, 1)


_glueb12_tc = pl.pallas_call(
    _glueb12_body,
    grid=(NBN,),
    in_specs=[
        pl.BlockSpec((NC, BN, 16), lambda i: (0, i, 0)),
        pl.BlockSpec((NC, BN, 16), lambda i: (0, i, 0)),
        pl.BlockSpec((NC, BN, 1), lambda i: (0, i, 0)),
        pl.BlockSpec((1, H), lambda i: (0, 0)),
        pl.BlockSpec((1, H), lambda i: (0, 0)),
        pl.BlockSpec((1, H), lambda i: (0, 0)),
        pl.BlockSpec((H, H), lambda i: (0, 0)),
        pl.BlockSpec((1, H), lambda i: (0, 0)),
        pl.BlockSpec((1, H), lambda i: (0, 0)),
    ],
    out_specs=[
        pl.BlockSpec((BN, 16), lambda i: (i, 0)),
        pl.BlockSpec((BN, 16), lambda i: (i, 0)),
        pl.BlockSpec((BN, 1), lambda i: (i, 0)),
        pl.BlockSpec((BN, 1), lambda i: (i, 0)),
        pl.BlockSpec((1, 1, 1), lambda i: (i, 0, 0)),
        pl.BlockSpec((1, 1, 1), lambda i: (i, 0, 0)),
    ],
    out_shape=[
        jax.ShapeDtypeStruct((N, 16), f32),
        jax.ShapeDtypeStruct((N, 16), f32),
        jax.ShapeDtypeStruct((N, 1), f32),
        jax.ShapeDtypeStruct((N, 1), f32),
        jax.ShapeDtypeStruct((NBN, 1, 1), f32),
        jax.ShapeDtypeStruct((NBN, 1, 1), f32),
    ],
    compiler_params=pltpu.CompilerParams(dimension_semantics=("arbitrary",)),
)


def _glueb23_body(alo_ref, ahi_ref, den_ref, bias_ref, gamma_ref, beta_ref,
                  w_ref, as_ref, ad_ref,
                  hw16_ref, ss_ref, sd_ref, ms_ref, md_ref):
    y = _combine_block(alo_ref, ahi_ref, den_ref, bias_ref, gamma_ref,
                       beta_ref)
    hw, ss, sd = _proj_block(y, w_ref, as_ref, ad_ref)
    hw16_ref[...] = jnp.concatenate(
        [hw, jnp.zeros((hw.shape[0], 16 - COUT), f32)], axis=1)
    ss_ref[...] = ss[:, None]
    sd_ref[...] = sd[:, None]
    ms_ref[...] = jnp.max(ss).reshape(1, 1, 1)
    md_ref[...] = jnp.max(sd).reshape(1, 1, 1)


_glueb23_tc = pl.pallas_call(
    _glueb23_body,
    grid=(NBN,),
    in_specs=[
        pl.BlockSpec((NC, BN, 16), lambda i: (0, i, 0)),
        pl.BlockSpec((NC, BN, 16), lambda i: (0, i, 0)),
        pl.BlockSpec((NC, BN, 1), lambda i: (0, i, 0)),
        pl.BlockSpec((1, H), lambda i: (0, 0)),
        pl.BlockSpec((1, H), lambda i: (0, 0)),
        pl.BlockSpec((1, H), lambda i: (0, 0)),
        pl.BlockSpec((H, COUT), lambda i: (0, 0)),
        pl.BlockSpec((1, COUT), lambda i: (0, 0)),
        pl.BlockSpec((1, COUT), lambda i: (0, 0)),
    ],
    out_specs=[
        pl.BlockSpec((BN, 16), lambda i: (i, 0)),
        pl.BlockSpec((BN, 1), lambda i: (i, 0)),
        pl.BlockSpec((BN, 1), lambda i: (i, 0)),
        pl.BlockSpec((1, 1, 1), lambda i: (i, 0, 0)),
        pl.BlockSpec((1, 1, 1), lambda i: (i, 0, 0)),
    ],
    out_shape=[
        jax.ShapeDtypeStruct((N, 16), f32),
        jax.ShapeDtypeStruct((N, 1), f32),
        jax.ShapeDtypeStruct((N, 1), f32),
        jax.ShapeDtypeStruct((NBN, 1, 1), f32),
        jax.ShapeDtypeStruct((NBN, 1, 1), f32),
    ],
    compiler_params=pltpu.CompilerParams(dimension_semantics=("arbitrary",)),
)


def _final_body(acc_ref, den_ref, bias_ref, out_ref):
    den = den_ref[0] + den_ref[1] + 1e-16
    num = acc_ref[0] + acc_ref[1]
    out_ref[...] = num[:, :COUT] / den + bias_ref[...]


_final_tc = pl.pallas_call(
    _final_body,
    grid=(NBN,),
    in_specs=[
        pl.BlockSpec((NC, BN, 16), lambda i: (0, i, 0)),
        pl.BlockSpec((NC, BN, 1), lambda i: (0, i, 0)),
        pl.BlockSpec((1, COUT), lambda i: (0, 0)),
    ],
    out_specs=pl.BlockSpec((BN, COUT), lambda i: (i, 0)),
    out_shape=jax.ShapeDtypeStruct((N, COUT), f32),
    compiler_params=pltpu.CompilerParams(dimension_semantics=("arbitrary",)),
)


def kernel(x, edge_index, edge_attr, W_ih, W_hh, b_ih, b_hh,
           W1, att_s1, att_d1, We1, att_e1, bias1, gamma1, beta1,
           W2, att_s2, att_d2, We2, att_e2, bias2, gamma2, beta2,
           W3, att_s3, att_d3, We3, att_e3, bias3):
    src = edge_index[0]
    dst = edge_index[1]

    xt = jnp.transpose(x, (1, 0, 2))
    bsum = (b_ih + b_hh).reshape(1, 4 * H)
    h0 = _lstm_tc(xt, W_ih.T, W_hh.T, bsum)

    cmat, cmax = _attr_tc(edge_attr.T, We1, att_e1.reshape(1, H),
                          We2, att_e2.reshape(1, H),
                          We3, att_e3.reshape(1, COUT))
    mc = jnp.max(cmax, axis=0).reshape(3)

    def mbound(ms, md, mcl):
        m = jnp.max(ms) + jnp.max(md) + mcl
        m = jnp.where(m < 0, m * 0.2, m)
        return jnp.full((16,), m, f32)

    lo, hi, ss, sd, ms, md = _gluea_tc(h0, W1, att_s1.reshape(1, H),
                                       att_d1.reshape(1, H))
    m1 = mbound(ms, md, mc[0])
    c1v = cmat[0]
    ss = ss.reshape(-1)
    sd = sd.reshape(-1)
    ea1, den = _make_pa3_ea()(src, dst, c1v, ss, sd, m1)
    acc_lo = _make_pa_rows()(src, dst, ea1, lo)[:, :N, :]
    acc_hi = _make_pa_rows()(src, dst, ea1, hi)[:, :N, :]
    den = den.reshape(NC, N, 1)

    lo, hi, ss, sd, ms, md = _glueb12_tc(
        acc_lo, acc_hi, den, bias1.reshape(1, H), gamma1.reshape(1, H),
        beta1.reshape(1, H), W2, att_s2.reshape(1, H), att_d2.reshape(1, H))
    m2 = mbound(ms, md, mc[1])
    c2v = cmat[1]
    ss = ss.reshape(-1)
    sd = sd.reshape(-1)
    ea2, den = _make_pa3_ea()(src, dst, c2v, ss, sd, m2)
    acc_lo = _make_pa_rows()(src, dst, ea2, lo)[:, :N, :]
    acc_hi = _make_pa_rows()(src, dst, ea2, hi)[:, :N, :]
    den = den.reshape(NC, N, 1)

    hw3, ss, sd, ms, md = _glueb23_tc(
        acc_lo, acc_hi, den, bias2.reshape(1, H), gamma2.reshape(1, H),
        beta2.reshape(1, H), W3, att_s3.reshape(1, COUT),
        att_d3.reshape(1, COUT))
    m3 = mbound(ms, md, mc[2])
    c3v = cmat[2]
    ss = ss.reshape(-1)
    sd = sd.reshape(-1)
    eav, den3 = _make_pa3_ea()(src, dst, c3v, ss, sd, m3)
    acc3 = _make_pa_rows()(src, dst, eav, hw3)[:, :N, :]

    return _final_tc(acc3, den3.reshape(NC, N, 1), bias3.reshape(1, COUT))
